# Initial kernel scaffold; baseline (speedup 1.0000x reference)
import jax
import jax.numpy as jnp
from jax.experimental import pallas as pl


def kernel(pred, target):
    raise NotImplementedError("write your pallas kernel here")


# speedup vs baseline: 2.1727x; 2.1727x over previous
import math

import jax
import jax.numpy as jnp
from jax.experimental import pallas as pl
from jax.experimental.pallas import tpu as pltpu

_C = 100000
_B = 1024
_SMOOTHING = 0.1
_CONF = 1.0 - _SMOOTHING
_S = _SMOOTHING / (_C - 1)
_W = 2048
_NBLK = (_C + _W - 1) // _W


def _loss_kernel(tgt_ref, x_ref, out_ref, m_ref, se_ref, rs_ref, g_ref):
    i = pl.program_id(0)

    @pl.when(i == 0)
    def _init():
        m_ref[...] = jnp.full_like(m_ref, -jnp.inf)
        se_ref[...] = jnp.zeros_like(se_ref)
        rs_ref[...] = jnp.zeros_like(rs_ref)
        g_ref[...] = jnp.zeros_like(g_ref)

    x = x_ref[...]
    col = jax.lax.broadcasted_iota(jnp.int32, x.shape, 1) + i * _W
    valid = col < _C
    xm = jnp.where(valid, x, -jnp.inf)
    x0 = jnp.where(valid, x, 0.0)

    m = m_ref[...]
    bm = jnp.max(xm, axis=1, keepdims=True)
    nm = jnp.maximum(m, bm)
    se_ref[...] = se_ref[...] * jnp.exp(m - nm) + jnp.sum(
        jnp.exp(xm - nm), axis=1, keepdims=True
    )
    m_ref[...] = nm
    rs_ref[...] += jnp.sum(x0, axis=1, keepdims=True)

    tgt = tgt_ref[...]
    hit = col == tgt
    g_ref[...] += jnp.sum(jnp.where(hit, x, 0.0), axis=1, keepdims=True)

    @pl.when(i == _NBLK - 1)
    def _fin():
        z = m_ref[...] + jnp.log(se_ref[...])
        zsum = jnp.sum(z)
        kc = _SMOOTHING * math.log(_S) + _CONF * math.log(_CONF)
        total = (
            _B * kc
            - _S * (jnp.sum(rs_ref[...]) - _C * zsum)
            - (_CONF - _S) * (jnp.sum(g_ref[...]) - zsum)
        )
        out_ref[0, 0] = total / (_B * _C)


def kernel(pred, target):
    tgt = target.astype(jnp.int32).reshape(_B, 1)
    out = pl.pallas_call(
        _loss_kernel,
        grid=(_NBLK,),
        in_specs=[
            pl.BlockSpec((_B, 1), lambda i: (0, 0)),
            pl.BlockSpec((_B, _W), lambda i: (0, i)),
        ],
        out_specs=pl.BlockSpec(
            (1, 1), lambda i: (0, 0), memory_space=pltpu.SMEM
        ),
        out_shape=jax.ShapeDtypeStruct((1, 1), jnp.float32),
        scratch_shapes=[
            pltpu.VMEM((_B, 1), jnp.float32),
            pltpu.VMEM((_B, 1), jnp.float32),
            pltpu.VMEM((_B, 1), jnp.float32),
            pltpu.VMEM((_B, 1), jnp.float32),
        ],
        compiler_params=pltpu.CompilerParams(
            dimension_semantics=("arbitrary",),
        ),
    )(tgt, pred)
    return out[0, 0]
